# baseline (speedup 1.0000x reference)
import functools

import jax
import jax.numpy as jnp
from jax import lax
from jax.experimental import pallas as pl
from jax.experimental.pallas import tpu as pltpu
from jax.experimental.pallas import tpu_sc as plsc

N = 16384
C = 1000
NC = 2
NS = 16
L = 16
NW = NC * NS
RW = N // NW
CHUNK = 16
NCHUNK = RW // CHUNK
UNROLL = 8


def _geo_body(cls_hbm, three_hbm, tr_hbm, sc_hbm, out_hbm,
              in_buf, tr_buf, sc_buf, three_buf, out_buf, sem0, sem1):
    cid = lax.axis_index("c")
    sid = lax.axis_index("s")
    wid = sid * NC + cid
    base = wid * RW

    pltpu.sync_copy(tr_hbm, tr_buf)
    pltpu.sync_copy(sc_hbm, sc_buf)
    pltpu.sync_copy(three_hbm.at[pl.ds(base * 3, RW * 3)], three_buf)

    lanes = lax.iota(jnp.int32, L)
    sems = (sem0, sem1)

    def start(k):
        return pltpu.async_copy(
            cls_hbm.at[pl.ds(base + k * CHUNK, CHUNK), :],
            in_buf.at[k % 2], sems[k % 2])

    pending = start(0)
    for k in range(NCHUNK):
        nxt = start(k + 1) if k + 1 < NCHUNK else None
        pending.wait()
        grp_ref = in_buf.at[k % 2]

        def gbody(j, carry):
            best, bidx = carry
            c0 = j * UNROLL
            for u in range(UNROLL):
                col = jnp.full((L,), c0 + u, jnp.int32)
                v = plsc.load_gather(grp_ref, [lanes, col])
                m = v > best
                best = jnp.where(m, v, best)
                bidx = jnp.where(m, col, bidx)
            return best, bidx

        init = (jnp.full((L,), -jnp.inf, jnp.float32),
                jnp.zeros((L,), jnp.int32))
        _, bidx = lax.fori_loop(0, C // UNROLL, gbody, init)

        rows3 = (lanes + k * CHUNK) * 3
        for d in range(3):
            dd = jnp.full((L,), d, jnp.int32)
            tr = plsc.load_gather(tr_buf, [dd, bidx])
            sc = plsc.load_gather(sc_buf, [dd, bidx])
            th = plsc.load_gather(three_buf, [rows3 + d])
            plsc.store_scatter(out_buf, [rows3 + d], th * sc + tr)
        pending = nxt

    pltpu.sync_copy(out_buf, out_hbm.at[pl.ds(base * 3, RW * 3)])


def kernel(class_pred, three_pred, geo_dict, translation, scale):
    del geo_dict
    mesh = plsc.VectorSubcoreMesh(core_axis_name="c", subcore_axis_name="s")
    f = functools.partial(
        pl.kernel,
        out_type=jax.ShapeDtypeStruct((N * 3,), jnp.float32),
        mesh=mesh,
        scratch_types=[
            pltpu.VMEM((2, CHUNK, C), jnp.float32),
            pltpu.VMEM((3, C), jnp.float32),
            pltpu.VMEM((3, C), jnp.float32),
            pltpu.VMEM((RW * 3,), jnp.float32),
            pltpu.VMEM((RW * 3,), jnp.float32),
            pltpu.SemaphoreType.DMA,
            pltpu.SemaphoreType.DMA,
        ],
        compiler_params=pltpu.CompilerParams(
            use_tc_tiling_on_sc=True, needs_layout_passes=False),
    )(_geo_body)
    out = f(class_pred, three_pred.reshape(N * 3), translation, scale)
    return out.reshape(N, 3)


# speedup vs baseline: 2.4519x; 2.4519x over previous
import functools

import jax
import jax.numpy as jnp
from jax import lax
from jax.experimental import pallas as pl
from jax.experimental.pallas import tpu as pltpu
from jax.experimental.pallas import tpu_sc as plsc

N = 16384
C = 1000
NC = 2
NS = 16
L = 16
NW = NC * NS
RW = N // NW
NGRP = RW // L

BR = 1024
NBLK = N // BR


def _argmax_body(x_ref, o_ref):
    x = x_ref[...]
    m = jnp.max(x, axis=1, keepdims=True)
    cols = lax.broadcasted_iota(jnp.int32, x.shape, 1)
    idx = jnp.min(jnp.where(x == m, cols, jnp.int32(C)), axis=1)
    o_ref[...] = idx


def _argmax(class_pred):
    return pl.pallas_call(
        _argmax_body,
        grid=(NBLK,),
        in_specs=[pl.BlockSpec((BR, C), lambda k: (k, 0))],
        out_specs=pl.BlockSpec((BR,), lambda k: (k,)),
        out_shape=jax.ShapeDtypeStruct((N,), jnp.int32),
    )(class_pred)


def _affine_body(cls_hbm, three_hbm, tr_hbm, sc_hbm, out_hbm,
                 cls_buf, tr_buf, sc_buf, three_buf, out_buf):
    cid = lax.axis_index("c")
    sid = lax.axis_index("s")
    wid = sid * NC + cid
    base = wid * RW

    pltpu.sync_copy(tr_hbm, tr_buf)
    pltpu.sync_copy(sc_hbm, sc_buf)
    pltpu.sync_copy(cls_hbm.at[pl.ds(base, RW)], cls_buf)
    pltpu.sync_copy(three_hbm.at[pl.ds(base * 3, RW * 3)], three_buf)

    lanes = lax.iota(jnp.int32, L)
    for g in range(NGRP):
        cls16 = cls_buf[pl.ds(g * L, L)]
        rows3 = (lanes + g * L) * 3
        for d in range(3):
            tr = plsc.load_gather(tr_buf, [cls16 + d * C])
            sc = plsc.load_gather(sc_buf, [cls16 + d * C])
            th = plsc.load_gather(three_buf, [rows3 + d])
            plsc.store_scatter(out_buf, [rows3 + d], th * sc + tr)

    pltpu.sync_copy(out_buf, out_hbm.at[pl.ds(base * 3, RW * 3)])


def _affine(classes, three_flat, tr_flat, sc_flat):
    mesh = plsc.VectorSubcoreMesh(core_axis_name="c", subcore_axis_name="s")
    f = functools.partial(
        pl.kernel,
        out_type=jax.ShapeDtypeStruct((N * 3,), jnp.float32),
        mesh=mesh,
        scratch_types=[
            pltpu.VMEM((RW,), jnp.int32),
            pltpu.VMEM((3 * C,), jnp.float32),
            pltpu.VMEM((3 * C,), jnp.float32),
            pltpu.VMEM((RW * 3,), jnp.float32),
            pltpu.VMEM((RW * 3,), jnp.float32),
        ],
        compiler_params=pltpu.CompilerParams(needs_layout_passes=False),
    )(_affine_body)
    return f(classes, three_flat, tr_flat, sc_flat)


def kernel(class_pred, three_pred, geo_dict, translation, scale):
    del geo_dict
    classes = _argmax(class_pred)
    out = _affine(classes, three_pred.reshape(N * 3),
                  translation.reshape(3 * C), scale.reshape(3 * C))
    return out.reshape(N, 3)
